# baseline (speedup 1.0000x reference)
import functools

import jax
import jax.numpy as jnp
from jax.experimental import pallas as pl


def _ce_body(conf_ref, tgt_ref, locp_ref, loct_ref, ce_ref, sl1_ref, *, rows, C):
    x = conf_ref[...]
    m = jnp.max(x, axis=1, keepdims=True)
    e = jnp.exp(x - m)
    s = jnp.sum(e, axis=1, keepdims=True)
    lse = jnp.log(s) + m
    t = tgt_ref[...]
    iot = jax.lax.broadcasted_iota(jnp.int32, (rows, C), 1)
    picked = jnp.sum(jnp.where(iot == t, x, 0.0), axis=1, keepdims=True)
    ce_ref[...] = lse - picked

    d = locp_ref[...] - loct_ref[...]
    ad = jnp.abs(d)
    sl1 = jnp.where(ad < 1.0, 0.5 * d * d, ad - 0.5)
    sl1_ref[...] = jnp.sum(sl1, axis=1, keepdims=True)


def _mine_body(ce_ref, sl1_ref, tgt_ref, out_ref, *, B, N):
    tgt = tgt_ref[...]
    pos = tgt > 0
    posf = pos.astype(jnp.float32)
    num_pos = jnp.sum(pos.astype(jnp.int32), axis=1, keepdims=True)
    num_matched = jnp.sum(posf)

    ce = ce_ref[...]
    sl1 = sl1_ref[...]
    loc_sum = jnp.sum(sl1 * posf)
    ce_pos_sum = jnp.sum(ce * posf)

    ce_sel = jnp.where(pos, 0.0, jnp.maximum(ce, 0.0))
    bits = jax.lax.bitcast_convert_type(ce_sel, jnp.int32)

    k = jnp.minimum(3 * num_pos, N - 1)

    def body(i, T):
        bit = 30 - i
        cand = T | jnp.left_shift(jnp.int32(1), bit)
        cnt = jnp.sum((bits >= cand).astype(jnp.int32), axis=1, keepdims=True)
        return jnp.where(cnt >= k, cand, T)

    T = jax.lax.fori_loop(0, 31, body, jnp.zeros((B, 1), jnp.int32))
    thr = jax.lax.bitcast_convert_type(T, jnp.float32)

    gt = bits > T
    cnt_gt = jnp.sum(gt.astype(jnp.int32), axis=1, keepdims=True)
    s_gt = jnp.sum(jnp.where(gt, ce_sel, 0.0), axis=1, keepdims=True)
    s_neg = s_gt + (k - cnt_gt).astype(jnp.float32) * thr
    s_neg = jnp.where(k > 0, s_neg, 0.0)

    total = loc_sum + ce_pos_sum + jnp.sum(s_neg)
    out_ref[...] = jnp.reshape(total / num_matched, (1, 1))


def kernel(loc_preds, loc_targets, conf_preds, conf_targets):
    B, N, _ = loc_preds.shape
    C = conf_preds.shape[-1]
    BN = B * N

    conf2 = conf_preds.reshape(BN, C)
    tgt2 = conf_targets.astype(jnp.int32).reshape(BN, 1)
    locp2 = loc_preds.reshape(BN, 4)
    loct2 = loc_targets.reshape(BN, 4)

    R = 4096
    grid = pl.cdiv(BN, R)
    ce, sl1 = pl.pallas_call(
        functools.partial(_ce_body, rows=R, C=C),
        grid=(grid,),
        in_specs=[
            pl.BlockSpec((R, C), lambda i: (i, 0)),
            pl.BlockSpec((R, 1), lambda i: (i, 0)),
            pl.BlockSpec((R, 4), lambda i: (i, 0)),
            pl.BlockSpec((R, 4), lambda i: (i, 0)),
        ],
        out_specs=[
            pl.BlockSpec((R, 1), lambda i: (i, 0)),
            pl.BlockSpec((R, 1), lambda i: (i, 0)),
        ],
        out_shape=[
            jax.ShapeDtypeStruct((BN, 1), jnp.float32),
            jax.ShapeDtypeStruct((BN, 1), jnp.float32),
        ],
    )(conf2, tgt2, locp2, loct2)

    tgtBN = conf_targets.astype(jnp.int32).reshape(B, N)
    out = pl.pallas_call(
        functools.partial(_mine_body, B=B, N=N),
        out_shape=jax.ShapeDtypeStruct((1, 1), jnp.float32),
    )(ce.reshape(B, N), sl1.reshape(B, N), tgtBN)
    return out[0, 0]


# speedup vs baseline: 20.6870x; 20.6870x over previous
import functools

import jax
import jax.numpy as jnp
from jax.experimental import pallas as pl


def _ce_body(conf_ref, tgt_ref, locp_ref, loct_ref, ce_ref, sl1_ref, *, C, N):
    x = conf_ref[0]
    m = jnp.max(x, axis=0, keepdims=True)
    e = jnp.exp(x - m)
    lse = jnp.log(jnp.sum(e, axis=0, keepdims=True)) + m
    t = tgt_ref[0]
    iot = jax.lax.broadcasted_iota(jnp.int32, (C, N), 0)
    picked = jnp.sum(jnp.where(iot == t, x, 0.0), axis=0, keepdims=True)
    ce_ref[0] = lse - picked

    d = locp_ref[0] - loct_ref[0]
    ad = jnp.abs(d)
    sl1 = jnp.where(ad < 1.0, 0.5 * d * d, ad - 0.5)
    sl1_ref[0] = jnp.sum(sl1, axis=0, keepdims=True)


def _mine_body(ce_ref, sl1_ref, tgt_ref, out_ref, *, B, N):
    tgt = tgt_ref[...]
    pos = tgt > 0
    posf = pos.astype(jnp.float32)
    num_pos = jnp.sum(pos.astype(jnp.int32), axis=1, keepdims=True)
    num_matched = jnp.sum(posf)

    ce = ce_ref[:, 0, :]
    sl1 = sl1_ref[:, 0, :]
    loc_sum = jnp.sum(sl1 * posf)
    ce_pos_sum = jnp.sum(ce * posf)

    ce_sel = jnp.where(pos, 0.0, jnp.maximum(ce, 0.0))
    bits = jax.lax.bitcast_convert_type(ce_sel, jnp.int32)

    k = jnp.minimum(3 * num_pos, N - 1)

    def body(i, T):
        bit = 30 - i
        cand = T | jnp.left_shift(jnp.int32(1), bit)
        cnt = jnp.sum((bits >= cand).astype(jnp.int32), axis=1, keepdims=True)
        return jnp.where(cnt >= k, cand, T)

    T = jax.lax.fori_loop(0, 31, body, jnp.zeros((B, 1), jnp.int32))
    thr = jax.lax.bitcast_convert_type(T, jnp.float32)

    gt = bits > T
    cnt_gt = jnp.sum(gt.astype(jnp.int32), axis=1, keepdims=True)
    s_gt = jnp.sum(jnp.where(gt, ce_sel, 0.0), axis=1, keepdims=True)
    s_neg = s_gt + (k - cnt_gt).astype(jnp.float32) * thr
    s_neg = jnp.where(k > 0, s_neg, 0.0)

    total = loc_sum + ce_pos_sum + jnp.sum(s_neg)
    out_ref[...] = jnp.reshape(total / num_matched, (1, 1))


def kernel(loc_preds, loc_targets, conf_preds, conf_targets):
    B, N, _ = loc_preds.shape
    C = conf_preds.shape[-1]

    conf_t = jnp.swapaxes(conf_preds, 1, 2)
    locp_t = jnp.swapaxes(loc_preds, 1, 2)
    loct_t = jnp.swapaxes(loc_targets, 1, 2)
    tgt = conf_targets.astype(jnp.int32)
    tgt3 = tgt.reshape(B, 1, N)

    ce, sl1 = pl.pallas_call(
        functools.partial(_ce_body, C=C, N=N),
        grid=(B,),
        in_specs=[
            pl.BlockSpec((1, C, N), lambda i: (i, 0, 0)),
            pl.BlockSpec((1, 1, N), lambda i: (i, 0, 0)),
            pl.BlockSpec((1, 4, N), lambda i: (i, 0, 0)),
            pl.BlockSpec((1, 4, N), lambda i: (i, 0, 0)),
        ],
        out_specs=[
            pl.BlockSpec((1, 1, N), lambda i: (i, 0, 0)),
            pl.BlockSpec((1, 1, N), lambda i: (i, 0, 0)),
        ],
        out_shape=[
            jax.ShapeDtypeStruct((B, 1, N), jnp.float32),
            jax.ShapeDtypeStruct((B, 1, N), jnp.float32),
        ],
    )(conf_t, tgt3, locp_t, loct_t)

    out = pl.pallas_call(
        functools.partial(_mine_body, B=B, N=N),
        out_shape=jax.ShapeDtypeStruct((1, 1), jnp.float32),
    )(ce, sl1, tgt)
    return out[0, 0]
